# baseline (speedup 1.0000x reference)
import functools

import jax
import jax.numpy as jnp
from jax import lax
from jax.experimental import pallas as pl
from jax.experimental.pallas import tpu as pltpu
from jax.experimental.pallas import tpu_sc as plsc

N = 10000
E = 320000
D = 128

NC = 2
NS = 16
NW = NC * NS
EPW = E // NW
CH = 50
NCHUNK = EPW // CH
NCPAD = NCHUNK + 4
PADR = 16
RPT = N // NS
RCH = 5
ZCH = 125

DEG_ROWS = 640

_sc_mesh = functools.partial(
    plsc.VectorSubcoreMesh, core_axis_name="c", subcore_axis_name="s")
_sc_params = pltpu.CompilerParams(
    needs_layout_passes=False, use_tc_tiling_on_sc=False)


def _deg_body(dst_hbm, out_hbm, dst_v, acc_v):
  wid = lax.axis_index("s") * NC + lax.axis_index("c")
  pltpu.sync_copy(dst_hbm.at[wid], dst_v)

  def zero(i, _):
    acc_v[pl.ds(i * 16, 16)] = jnp.zeros((16,), jnp.float32)
    return 0
  lax.fori_loop(0, DEG_ROWS, zero, 0)

  ones = jnp.ones((16,), jnp.float32)

  def acc(i, _):
    d = dst_v[i, :]
    plsc.addupdate_scatter(acc_v, [d], ones)
    return 0
  lax.fori_loop(0, EPW // 16, acc, 0)

  pltpu.sync_copy(acc_v, out_hbm.at[wid])


def _deg_partials(dst):
  k = pl.kernel(
      _deg_body,
      out_type=jax.ShapeDtypeStruct((NW, DEG_ROWS * 16), jnp.float32),
      mesh=_sc_mesh(),
      scratch_types=[
          pltpu.VMEM((EPW // 16, 16), jnp.int32),
          pltpu.VMEM((DEG_ROWS * 16,), jnp.float32),
      ],
      compiler_params=_sc_params,
  )
  return k(dst)


def _prop_body(g_hbm, src_hbm, dst_hbm, out_hbm, src_v, dst_v,
               b0, b1, b2, b3, acc_sh, g0, g1, g2, g3, s0, s1, s2, s3):
  c = lax.axis_index("c")
  s = lax.axis_index("s")
  wid = s * NC + c
  pltpu.sync_copy(src_hbm.at[wid], src_v)
  pltpu.sync_copy(dst_hbm.at[wid], dst_v)

  def zero(i, _):
    for b in range(D // 16):
      b0[i, pl.ds(b * 16, 16)] = jnp.zeros((16,), jnp.float32)
    return 0
  lax.fori_loop(0, CH, zero, 0)
  for k in range(12):
    pltpu.sync_copy(b0, acc_sh.at[pl.ds(s * RPT + k * CH, CH)])
  pltpu.sync_copy(b0.at[pl.ds(0, RPT - 12 * CH)],
                  acc_sh.at[pl.ds(s * RPT + 12 * CH, RPT - 12 * CH)])
  plsc.subcore_barrier()

  def gwait(buf, sem):
    pltpu.make_async_copy(g_hbm.at[src_v.at[0]], buf, sem).wait()

  def swait(buf, sem):
    pltpu.make_async_copy(buf, acc_sh.at[dst_v.at[0]], sem).wait()

  hp0 = pltpu.async_copy(g_hbm.at[src_v.at[0]], b2, g2)
  hp1 = pltpu.async_copy(g_hbm.at[src_v.at[1]], b3, g3)
  hp0.wait()
  hp1.wait()
  pltpu.async_copy(g_hbm.at[src_v.at[2]], b0, g0)
  pltpu.async_copy(g_hbm.at[src_v.at[3]], b1, g1)
  pltpu.async_copy(b2, acc_sh.at[dst_v.at[0]], s2, add=True)
  pltpu.async_copy(b3, acc_sh.at[dst_v.at[1]], s3, add=True)

  def step(u, _):
    base = 2 + 4 * u
    gwait(b0, g0)
    gwait(b1, g1)
    swait(b2, s2)
    swait(b3, s3)
    hb2 = pltpu.async_copy(g_hbm.at[src_v.at[base + 2]], b2, g2)
    hb3 = pltpu.async_copy(g_hbm.at[src_v.at[base + 3]], b3, g3)
    hs0 = pltpu.async_copy(b0, acc_sh.at[dst_v.at[base]], s0, add=True)
    hs1 = pltpu.async_copy(b1, acc_sh.at[dst_v.at[base + 1]], s1, add=True)
    hb2.wait()
    hb3.wait()
    hs0.wait()
    hs1.wait()
    pltpu.async_copy(g_hbm.at[src_v.at[base + 4]], b0, g0)
    pltpu.async_copy(g_hbm.at[src_v.at[base + 5]], b1, g1)
    pltpu.async_copy(b2, acc_sh.at[dst_v.at[base + 2]], s2, add=True)
    pltpu.async_copy(b3, acc_sh.at[dst_v.at[base + 3]], s3, add=True)
    return 0
  lax.fori_loop(0, NCHUNK // 4, step, 0)

  gwait(b0, g0)
  gwait(b1, g1)
  swait(b2, s2)
  swait(b3, s3)
  plsc.subcore_barrier()

  pltpu.sync_copy(acc_sh.at[pl.ds(s * RPT, RPT)],
                  out_hbm.at[c].at[pl.ds(s * RPT, RPT)])


def _propagate(g, src, dst):
  k = pl.kernel(
      _prop_body,
      out_type=jax.ShapeDtypeStruct((NC, N, D), jnp.float32),
      mesh=_sc_mesh(),
      scratch_types=[
          pltpu.VMEM((NCPAD, CH), jnp.int32),
          pltpu.VMEM((NCPAD, CH), jnp.int32),
          pltpu.VMEM((CH, D), jnp.float32),
          pltpu.VMEM((CH, D), jnp.float32),
          pltpu.VMEM((CH, D), jnp.float32),
          pltpu.VMEM((CH, D), jnp.float32),
          pltpu.VMEM_SHARED((N, D), jnp.float32),
      ] + [pltpu.SemaphoreType.DMA] * 8,
      compiler_params=_sc_params,
  )
  return k(g, src, dst)


def _tc1_body(x_ref, w_ref, dinv_ref, o_ref):
  h = jnp.dot(x_ref[...], w_ref[...], preferred_element_type=jnp.float32)
  o_ref[0:N, :] = h * dinv_ref[...]
  o_ref[N:N + PADR, :] = jnp.zeros((PADR, D), jnp.float32)


def _tc2_body(p_ref, g_ref, dinv_ref, b_ref, w_ref, o_ref):
  agg = (p_ref[0] + p_ref[1] + g_ref[0:N, :]) * dinv_ref[...] + b_ref[...]
  h1 = jnp.maximum(agg, 0.0)
  h2 = jnp.dot(h1, w_ref[...], preferred_element_type=jnp.float32)
  o_ref[0:N, :] = h2 * dinv_ref[...]
  o_ref[N:N + PADR, :] = jnp.zeros((PADR, D), jnp.float32)


def _tc3_body(p_ref, g_ref, dinv_ref, b_ref, o_ref):
  o_ref[...] = (p_ref[0] + p_ref[1] + g_ref[0:N, :]) * dinv_ref[...] + b_ref[...]


def _tc_call(body, nrows, *args):
  return pl.pallas_call(
      body, out_shape=jax.ShapeDtypeStruct((nrows, D), jnp.float32))(*args)


@jax.jit
def kernel(x, edge_index, W1, b1, W2, b2):
  pad_s = jnp.full((NW, 2, CH), N, jnp.int32)
  pad_d = jnp.zeros((NW, 2, CH), jnp.int32)
  src = jnp.concatenate(
      [pad_s, edge_index[0].reshape(NW, NCHUNK, CH), pad_s], axis=1)
  dst = jnp.concatenate(
      [pad_d, edge_index[1].reshape(NW, NCHUNK, CH), pad_d], axis=1)
  dst16 = edge_index[1].reshape(NW, EPW // 16, 16)

  degp = _deg_partials(dst16)
  deg = 1.0 + jnp.sum(degp, axis=0)[:N]
  dinv = lax.rsqrt(deg).reshape(N, 1)

  g1 = _tc_call(_tc1_body, N + PADR, x, W1, dinv)
  p1 = _propagate(g1, src, dst)
  g2 = _tc_call(_tc2_body, N + PADR, p1, g1, dinv, b1.reshape(1, D), W2)
  p2 = _propagate(g2, src, dst)
  out = _tc_call(_tc3_body, N, p2, g2, dinv, b2.reshape(1, D))
  return out


# speedup vs baseline: 2.6331x; 2.6331x over previous
import functools

import jax
import jax.numpy as jnp
from jax import lax
from jax.experimental import pallas as pl
from jax.experimental.pallas import tpu as pltpu
from jax.experimental.pallas import tpu_sc as plsc

N = 10000
E = 320000
D = 128

NC = 2
NS = 16
NW = NC * NS
EPW = E // NW
CH = 100
NCHUNK = EPW // CH
NBUF = 2
UNROLL = 10
RPT = N // NS
RCH = 5
ZCH = 125

DEG_ROWS = 640

_sc_mesh = functools.partial(
    plsc.VectorSubcoreMesh, core_axis_name="c", subcore_axis_name="s")
_sc_params = pltpu.CompilerParams(
    needs_layout_passes=False, use_tc_tiling_on_sc=False)


def _deg_body(dst_hbm, out_hbm, dst_v, acc_v):
  wid = lax.axis_index("s") * NC + lax.axis_index("c")
  pltpu.sync_copy(dst_hbm.at[wid], dst_v)

  def zero(i, _):
    acc_v[pl.ds(i * 16, 16)] = jnp.zeros((16,), jnp.float32)
    return 0
  lax.fori_loop(0, DEG_ROWS, zero, 0)

  ones = jnp.ones((16,), jnp.float32)

  def acc(i, _):
    d = dst_v[i, :]
    plsc.addupdate_scatter(acc_v, [d], ones)
    return 0
  lax.fori_loop(0, EPW // 16, acc, 0)

  pltpu.sync_copy(acc_v, out_hbm.at[wid])


def _deg_partials(dst):
  k = pl.kernel(
      _deg_body,
      out_type=jax.ShapeDtypeStruct((NW, DEG_ROWS * 16), jnp.float32),
      mesh=_sc_mesh(),
      scratch_types=[
          pltpu.VMEM((EPW // 16, 16), jnp.int32),
          pltpu.VMEM((DEG_ROWS * 16,), jnp.float32),
      ],
      compiler_params=_sc_params,
  )
  return k(dst)


def _prop_body(g_hbm, src_hbm, dst_hbm, out_hbm, src_v, dst_v,
               r0_v, r1_v, acc_sh, g0, g1, s0, s1):
  rows = (r0_v, r1_v)
  gsem = (g0, g1)
  ssem = (s0, s1)
  c = lax.axis_index("c")
  s = lax.axis_index("s")
  wid = s * NC + c
  pltpu.sync_copy(src_hbm.at[wid], src_v)
  pltpu.sync_copy(dst_hbm.at[wid], dst_v)

  def zero(i, _):
    for b in range(D // 16):
      r0_v[i, pl.ds(b * 16, 16)] = jnp.zeros((16,), jnp.float32)
    return 0
  lax.fori_loop(0, CH, zero, 0)
  for k in range(6):
    pltpu.sync_copy(r0_v, acc_sh.at[pl.ds(s * RPT + k * CH, CH)])
  pltpu.sync_copy(r0_v.at[pl.ds(0, RPT - 6 * CH)],
                  acc_sh.at[pl.ds(s * RPT + 6 * CH, RPT - 6 * CH)])
  plsc.subcore_barrier()

  def step(t, _):
    j = t * UNROLL
    hg = pltpu.async_copy(g_hbm.at[src_v.at[j]], rows[0], gsem[0])
    hs = None
    for i in range(UNROLL):
      if hs is not None:
        hs.wait()
      hg_next = None
      if i + 1 < UNROLL:
        hg_next = pltpu.async_copy(
            g_hbm.at[src_v.at[j + i + 1]], rows[(i + 1) % 2], gsem[(i + 1) % 2])
      hg.wait()
      hs = pltpu.async_copy(rows[i % 2], acc_sh.at[dst_v.at[j + i]],
                            ssem[0], add=True)
      hg = hg_next
    hs.wait()
    return 0
  lax.fori_loop(0, NCHUNK // UNROLL, step, 0)
  plsc.subcore_barrier()

  pltpu.sync_copy(acc_sh.at[pl.ds(s * RPT, RPT)],
                  out_hbm.at[c].at[pl.ds(s * RPT, RPT)])


def _propagate(g, src, dst):
  k = pl.kernel(
      _prop_body,
      out_type=jax.ShapeDtypeStruct((NC, N, D), jnp.float32),
      mesh=_sc_mesh(),
      scratch_types=[
          pltpu.VMEM((NCHUNK, CH), jnp.int32),
          pltpu.VMEM((NCHUNK, CH), jnp.int32),
          pltpu.VMEM((CH, D), jnp.float32),
          pltpu.VMEM((CH, D), jnp.float32),
          pltpu.VMEM_SHARED((N, D), jnp.float32),
      ] + [pltpu.SemaphoreType.DMA] * (2 * NBUF),
      compiler_params=_sc_params,
  )
  return k(g, src, dst)


def _tc1_body(x_ref, w_ref, dinv_ref, o_ref):
  h = jnp.dot(x_ref[...], w_ref[...], preferred_element_type=jnp.float32)
  o_ref[...] = h * dinv_ref[...]


def _tc2_body(p_ref, g_ref, dinv_ref, b_ref, w_ref, o_ref):
  agg = (p_ref[0] + p_ref[1] + g_ref[...]) * dinv_ref[...] + b_ref[...]
  h1 = jnp.maximum(agg, 0.0)
  h2 = jnp.dot(h1, w_ref[...], preferred_element_type=jnp.float32)
  o_ref[...] = h2 * dinv_ref[...]


def _tc3_body(p_ref, g_ref, dinv_ref, b_ref, o_ref):
  o_ref[...] = (p_ref[0] + p_ref[1] + g_ref[...]) * dinv_ref[...] + b_ref[...]


def _tc_call(body, *args):
  return pl.pallas_call(
      body, out_shape=jax.ShapeDtypeStruct((N, D), jnp.float32))(*args)


@jax.jit
def kernel(x, edge_index, W1, b1, W2, b2):
  src = edge_index[0].reshape(NW, NCHUNK, CH)
  dst = edge_index[1].reshape(NW, NCHUNK, CH)
  dst16 = edge_index[1].reshape(NW, EPW // 16, 16)

  degp = _deg_partials(dst16)
  deg = 1.0 + jnp.sum(degp, axis=0)[:N]
  dinv = lax.rsqrt(deg).reshape(N, 1)

  g1 = _tc_call(_tc1_body, x, W1, dinv)
  p1 = _propagate(g1, src, dst)
  g2 = _tc_call(_tc2_body, p1, g1, dinv, b1.reshape(1, D), W2)
  p2 = _propagate(g2, src, dst)
  out = _tc_call(_tc3_body, p2, g2, dinv, b2.reshape(1, D))
  return out


# speedup vs baseline: 2.7178x; 1.0322x over previous
import functools

import jax
import jax.numpy as jnp
from jax import lax
from jax.experimental import pallas as pl
from jax.experimental.pallas import tpu as pltpu
from jax.experimental.pallas import tpu_sc as plsc

N = 10000
E = 320000
D = 128

NC = 2
NS = 16
NW = NC * NS
EPW = E // NW
CH = 100
NCHUNK = EPW // CH
NBUF = 2
UNROLL = 20
RPT = N // NS
RCH = 5
ZCH = 125

DEG_ROWS = 640

_sc_mesh = functools.partial(
    plsc.VectorSubcoreMesh, core_axis_name="c", subcore_axis_name="s")
_sc_params = pltpu.CompilerParams(
    needs_layout_passes=False, use_tc_tiling_on_sc=False)


def _deg_body(dst_hbm, out_hbm, dst_v, acc_v):
  wid = lax.axis_index("s") * NC + lax.axis_index("c")
  pltpu.sync_copy(dst_hbm.at[wid], dst_v)

  def zero(i, _):
    acc_v[pl.ds(i * 16, 16)] = jnp.zeros((16,), jnp.float32)
    return 0
  lax.fori_loop(0, DEG_ROWS, zero, 0)

  ones = jnp.ones((16,), jnp.float32)

  def acc(i, _):
    d = dst_v[i, :]
    plsc.addupdate_scatter(acc_v, [d], ones)
    return 0
  lax.fori_loop(0, EPW // 16, acc, 0)

  pltpu.sync_copy(acc_v, out_hbm.at[wid])


def _deg_partials(dst):
  k = pl.kernel(
      _deg_body,
      out_type=jax.ShapeDtypeStruct((NW, DEG_ROWS * 16), jnp.float32),
      mesh=_sc_mesh(),
      scratch_types=[
          pltpu.VMEM((EPW // 16, 16), jnp.int32),
          pltpu.VMEM((DEG_ROWS * 16,), jnp.float32),
      ],
      compiler_params=_sc_params,
  )
  return k(dst)


def _prop_body(g_hbm, src_hbm, dst_hbm, out_hbm, src_v, dst_v,
               r0_v, r1_v, acc_sh, g0, g1, s0, s1):
  rows = (r0_v, r1_v)
  gsem = (g0, g1)
  ssem = (s0, s1)
  c = lax.axis_index("c")
  s = lax.axis_index("s")
  wid = s * NC + c
  pltpu.sync_copy(src_hbm.at[wid], src_v)
  pltpu.sync_copy(dst_hbm.at[wid], dst_v)

  def zero(i, _):
    for b in range(D // 16):
      r0_v[i, pl.ds(b * 16, 16)] = jnp.zeros((16,), jnp.float32)
    return 0
  lax.fori_loop(0, CH, zero, 0)
  for k in range(6):
    pltpu.sync_copy(r0_v, acc_sh.at[pl.ds(s * RPT + k * CH, CH)])
  pltpu.sync_copy(r0_v.at[pl.ds(0, RPT - 6 * CH)],
                  acc_sh.at[pl.ds(s * RPT + 6 * CH, RPT - 6 * CH)])
  plsc.subcore_barrier()

  def step(t, _):
    j = t * UNROLL
    hg = pltpu.async_copy(g_hbm.at[src_v.at[j]], rows[0], gsem[0])
    hs = None
    for i in range(UNROLL):
      if hs is not None:
        hs.wait()
      hg_next = None
      if i + 1 < UNROLL:
        hg_next = pltpu.async_copy(
            g_hbm.at[src_v.at[j + i + 1]], rows[(i + 1) % 2], gsem[(i + 1) % 2])
      hg.wait()
      hs = pltpu.async_copy(rows[i % 2], acc_sh.at[dst_v.at[j + i]],
                            ssem[0], add=True)
      hg = hg_next
    hs.wait()
    return 0
  lax.fori_loop(0, NCHUNK // UNROLL, step, 0)
  plsc.subcore_barrier()

  pltpu.sync_copy(acc_sh.at[pl.ds(s * RPT, RPT)],
                  out_hbm.at[c].at[pl.ds(s * RPT, RPT)])


def _propagate(g, src, dst):
  k = pl.kernel(
      _prop_body,
      out_type=jax.ShapeDtypeStruct((NC, N, D), jnp.float32),
      mesh=_sc_mesh(),
      scratch_types=[
          pltpu.VMEM((NCHUNK, CH), jnp.int32),
          pltpu.VMEM((NCHUNK, CH), jnp.int32),
          pltpu.VMEM((CH, D), jnp.float32),
          pltpu.VMEM((CH, D), jnp.float32),
          pltpu.VMEM_SHARED((N, D), jnp.float32),
      ] + [pltpu.SemaphoreType.DMA] * (2 * NBUF),
      compiler_params=_sc_params,
  )
  return k(g, src, dst)


def _tc1_body(x_ref, w_ref, dinv_ref, o_ref):
  h = jnp.dot(x_ref[...], w_ref[...], preferred_element_type=jnp.float32)
  o_ref[...] = h * dinv_ref[...]


def _tc2_body(p_ref, g_ref, dinv_ref, b_ref, w_ref, o_ref):
  agg = (p_ref[0] + p_ref[1] + g_ref[...]) * dinv_ref[...] + b_ref[...]
  h1 = jnp.maximum(agg, 0.0)
  h2 = jnp.dot(h1, w_ref[...], preferred_element_type=jnp.float32)
  o_ref[...] = h2 * dinv_ref[...]


def _tc3_body(p_ref, g_ref, dinv_ref, b_ref, o_ref):
  o_ref[...] = (p_ref[0] + p_ref[1] + g_ref[...]) * dinv_ref[...] + b_ref[...]


def _tc_call(body, *args):
  return pl.pallas_call(
      body, out_shape=jax.ShapeDtypeStruct((N, D), jnp.float32))(*args)


@jax.jit
def kernel(x, edge_index, W1, b1, W2, b2):
  src = edge_index[0].reshape(NW, NCHUNK, CH)
  dst = edge_index[1].reshape(NW, NCHUNK, CH)
  dst16 = edge_index[1].reshape(NW, EPW // 16, 16)

  degp = _deg_partials(dst16)
  deg = 1.0 + jnp.sum(degp, axis=0)[:N]
  dinv = lax.rsqrt(deg).reshape(N, 1)

  g1 = _tc_call(_tc1_body, x, W1, dinv)
  p1 = _propagate(g1, src, dst)
  g2 = _tc_call(_tc2_body, p1, g1, dinv, b1.reshape(1, D), W2)
  p2 = _propagate(g2, src, dst)
  out = _tc_call(_tc3_body, p2, g2, dinv, b2.reshape(1, D))
  return out


# speedup vs baseline: 2.7654x; 1.0175x over previous
import functools

import jax
import jax.numpy as jnp
from jax import lax
from jax.experimental import pallas as pl
from jax.experimental.pallas import tpu as pltpu
from jax.experimental.pallas import tpu_sc as plsc

N = 10000
E = 320000
D = 128

NC = 2
NS = 16
NW = NC * NS
EPW = E // NW
CH = 100
NCHUNK = EPW // CH
NBUF = 2
UNROLL = 20
RPT = N // NS
RCH = 5
ZCH = 125

DEG_ROWS = 640

_sc_mesh = functools.partial(
    plsc.VectorSubcoreMesh, core_axis_name="c", subcore_axis_name="s")
_sc_params = pltpu.CompilerParams(
    needs_layout_passes=False, use_tc_tiling_on_sc=False)


def _deg_body(dst_hbm, out_hbm, dst_v, acc_v):
  wid = lax.axis_index("s") * NC + lax.axis_index("c")
  pltpu.sync_copy(dst_hbm.at[wid], dst_v)

  def zero(i, _):
    acc_v[pl.ds(i * 16, 16)] = jnp.zeros((16,), jnp.float32)
    return 0
  lax.fori_loop(0, DEG_ROWS, zero, 0)

  ones = jnp.ones((16,), jnp.float32)

  def acc(i, _):
    d = dst_v[i, :]
    plsc.addupdate_scatter(acc_v, [d], ones)
    return 0
  lax.fori_loop(0, EPW // 16, acc, 0)

  pltpu.sync_copy(acc_v, out_hbm.at[wid])


def _deg_partials(dst):
  k = pl.kernel(
      _deg_body,
      out_type=jax.ShapeDtypeStruct((NW, DEG_ROWS * 16), jnp.float32),
      mesh=_sc_mesh(),
      scratch_types=[
          pltpu.VMEM((EPW // 16, 16), jnp.int32),
          pltpu.VMEM((DEG_ROWS * 16,), jnp.float32),
      ],
      compiler_params=_sc_params,
  )
  return k(dst)


def _prop_body(g_hbm, src_hbm, dst_hbm, out_hbm, src_v, dst_v,
               r0_v, r1_v, acc_sh, g0, g1, s0, s1):
  rows = (r0_v, r1_v)
  gsem = (g0, g1)
  ssem = (s0, s1)
  c = lax.axis_index("c")
  s = lax.axis_index("s")
  wid = s * NC + c
  hsrc = pltpu.async_copy(src_hbm.at[wid], src_v, s0)
  hdst = pltpu.async_copy(dst_hbm.at[wid], dst_v, s1)

  def zero(i, _):
    for b in range(D // 16):
      r0_v[i, pl.ds(b * 16, 16)] = jnp.zeros((16,), jnp.float32)
    return 0
  lax.fori_loop(0, CH, zero, 0)
  hz = [pltpu.async_copy(r0_v, acc_sh.at[pl.ds(s * RPT + k * CH, CH)],
                         gsem[k % 2]) for k in range(6)]
  hz.append(pltpu.async_copy(r0_v.at[pl.ds(0, RPT - 6 * CH)],
                             acc_sh.at[pl.ds(s * RPT + 6 * CH, RPT - 6 * CH)],
                             gsem[0]))
  for h in hz:
    h.wait()
  hsrc.wait()
  hdst.wait()
  plsc.subcore_barrier()

  def step(t, _):
    j = t * UNROLL
    hg = pltpu.async_copy(g_hbm.at[src_v.at[j]], rows[0], gsem[0])
    hs = None
    for i in range(UNROLL):
      if hs is not None:
        hs.wait()
      hg_next = None
      if i + 1 < UNROLL:
        hg_next = pltpu.async_copy(
            g_hbm.at[src_v.at[j + i + 1]], rows[(i + 1) % 2], gsem[(i + 1) % 2])
      hg.wait()
      hs = pltpu.async_copy(rows[i % 2], acc_sh.at[dst_v.at[j + i]],
                            ssem[0], add=True)
      hg = hg_next
    hs.wait()
    return 0
  lax.fori_loop(0, NCHUNK // UNROLL, step, 0)
  plsc.subcore_barrier()

  pltpu.sync_copy(acc_sh.at[pl.ds(s * RPT, RPT)],
                  out_hbm.at[c].at[pl.ds(s * RPT, RPT)])


def _propagate(g, src, dst):
  k = pl.kernel(
      _prop_body,
      out_type=jax.ShapeDtypeStruct((NC, N, D), jnp.float32),
      mesh=_sc_mesh(),
      scratch_types=[
          pltpu.VMEM((NCHUNK, CH), jnp.int32),
          pltpu.VMEM((NCHUNK, CH), jnp.int32),
          pltpu.VMEM((CH, D), jnp.float32),
          pltpu.VMEM((CH, D), jnp.float32),
          pltpu.VMEM_SHARED((N, D), jnp.float32),
      ] + [pltpu.SemaphoreType.DMA] * (2 * NBUF),
      compiler_params=_sc_params,
  )
  return k(g, src, dst)


def _tc1_body(x_ref, w_ref, dinv_ref, o_ref):
  h = jnp.dot(x_ref[...], w_ref[...], preferred_element_type=jnp.float32)
  o_ref[...] = h * dinv_ref[...]


def _tc2_body(p_ref, g_ref, dinv_ref, b_ref, w_ref, o_ref):
  agg = (p_ref[0] + p_ref[1] + g_ref[...]) * dinv_ref[...] + b_ref[...]
  h1 = jnp.maximum(agg, 0.0)
  h2 = jnp.dot(h1, w_ref[...], preferred_element_type=jnp.float32)
  o_ref[...] = h2 * dinv_ref[...]


def _tc3_body(p_ref, g_ref, dinv_ref, b_ref, o_ref):
  o_ref[...] = (p_ref[0] + p_ref[1] + g_ref[...]) * dinv_ref[...] + b_ref[...]


def _tc_call(body, *args):
  return pl.pallas_call(
      body, out_shape=jax.ShapeDtypeStruct((N, D), jnp.float32))(*args)


@jax.jit
def kernel(x, edge_index, W1, b1, W2, b2):
  src = edge_index[0].reshape(NW, NCHUNK, CH)
  dst = edge_index[1].reshape(NW, NCHUNK, CH)
  dst16 = edge_index[1].reshape(NW, EPW // 16, 16)

  degp = _deg_partials(dst16)
  deg = 1.0 + jnp.sum(degp, axis=0)[:N]
  dinv = lax.rsqrt(deg).reshape(N, 1)

  g1 = _tc_call(_tc1_body, x, W1, dinv)
  p1 = _propagate(g1, src, dst)
  g2 = _tc_call(_tc2_body, p1, g1, dinv, b1.reshape(1, D), W2)
  p2 = _propagate(g2, src, dst)
  out = _tc_call(_tc3_body, p2, g2, dinv, b2.reshape(1, D))
  return out
